# baseline (speedup 1.0000x reference)
import dataclasses
import functools

import jax
import jax.numpy as jnp
from jax import lax
from jax.experimental import pallas as pl
from jax.experimental.pallas import tpu as pltpu
from jax.experimental.pallas import tpu_sc as plsc

K = 16
D = 256


def _cm_body(xyzT_ref, sq_ref, xyzr_ref, sqr_ref, cm_ref):
    W = 2048
    RB = 256
    xb = xyzT_ref[0].astype(jnp.bfloat16)
    xrows = xyzr_ref[0].astype(jnp.bfloat16)
    s = lax.dot_general(xrows, xb, (((0,), (0,)), ((), ())),
                        preferred_element_type=jnp.float32)
    sqv = sq_ref[0, 0]
    sqr = sqr_ref[0, 0]
    d2 = (sqr[:, None] + sqv[None, :]) - 2.0 * s
    cm_ref[0] = jnp.min(d2.reshape(RB, W // 16, 16), axis=-1)


def _chunkmin_tc(xyzT, sq):
    B, W = sq.shape
    RB = 256
    return pl.pallas_call(
        _cm_body,
        grid=(B, W // RB),
        in_specs=[
            pl.BlockSpec((1, 3, W), lambda b, i: (b, 0, 0)),
            pl.BlockSpec((1, 1, W), lambda b, i: (b, 0, 0)),
            pl.BlockSpec((1, 3, RB), lambda b, i: (b, 0, i)),
            pl.BlockSpec((1, 1, RB), lambda b, i: (b, 0, i)),
        ],
        out_specs=pl.BlockSpec((1, RB, W // 16), lambda b, i: (b, i, 0)),
        out_shape=jax.ShapeDtypeStruct((B, W, W // 16), jnp.float32),
    )(xyzT.reshape(B, 3, W), sq.reshape(B, 1, W),
      xyzT.reshape(B, 3, W), sq.reshape(B, 1, W)).reshape(B, W * (W // 16))


def _topk_sc_body(xyz_hbm, sq_hbm, cm_hbm, out_hbm, xv, yv, zv, sqv,
                  cmbuf, cmivm, cmvvm, idxbuf):
    W = 2048
    ROWS = 512
    wid = lax.axis_index("s") * 2 + lax.axis_index("c")
    b = wid // 4
    row0 = (wid % 4) * ROWS
    pltpu.sync_copy(xyz_hbm.at[b * 3 + 0], xv)
    pltpu.sync_copy(xyz_hbm.at[b * 3 + 1], yv)
    pltpu.sync_copy(xyz_hbm.at[b * 3 + 2], zv)
    pltpu.sync_copy(sq_hbm.at[b], sqv)
    pltpu.sync_copy(cm_hbm.at[b, pl.ds(row0 * 128, ROWS * 128)], cmbuf)
    iota = lax.iota(jnp.int32, 16)

    @pl.loop(0, W // 16)
    def _rnd(c):
        s = pl.ds(c * 16, 16)
        for ref in (xv, yv, zv):
            u = plsc.bitcast(ref[s], jnp.int32)
            u = (u + 0x7FFF + ((u >> 16) & 1)) & ~0xFFFF
            ref[s] = plsc.bitcast(u, jnp.float32)

    def _merge(d2v, midx, rv, ri):
        cv, ci = plsc.sort_key_val(d2v, midx)
        rcv = lax.rev(cv, (0,))
        rci = lax.rev(ci, (0,))
        m = rv <= rcv
        nv = jnp.where(m, rv, rcv)
        ni = jnp.where(m, ri, rci)
        rv2, ri2 = plsc.sort_key_val(nv, ni)
        t2 = jnp.full((16,), jnp.max(rv2))
        return rv2, ri2, t2

    inf16 = jnp.full((16,), jnp.inf, jnp.float32)
    zi16 = jnp.zeros((16,), jnp.int32)

    @pl.loop(0, ROWS)
    def _row(r):
        s0 = jnp.full((16,), row0 + r, jnp.int32)
        xn = plsc.load_gather(xv, [s0])
        yn = plsc.load_gather(yv, [s0])
        zn = plsc.load_gather(zv, [s0])
        qn = plsc.load_gather(sqv, [s0])

        def cstep(c, carry):
            rv, ri, t = carry
            cmc = cmbuf[pl.ds(r * 128 + c * 16, 16)]

            def m(rv, ri, t):
                return _merge(cmc, iota + c * 16, rv, ri)

            return lax.cond(jnp.any(cmc < t), m,
                            lambda a, b_, c_: (a, b_, c_), rv, ri, t)

        cmv16, cmi16, _ = lax.fori_loop(0, 8, cstep, (inf16, zi16, inf16))
        cmvvm[pl.ds(0, 16)] = cmv16
        cmivm[pl.ds(0, 16)] = cmi16

        def jstep(j, carry):
            rv, ri, t = carry
            js = jnp.full((16,), j, jnp.int32)
            cmin = plsc.load_gather(cmvvm, [js])

            def proc(rv, ri, t):
                cid = plsc.load_gather(cmivm, [js])
                colv = cid * 16 + iota
                xc = plsc.load_gather(xv, [colv])
                yc = plsc.load_gather(yv, [colv])
                zc = plsc.load_gather(zv, [colv])
                qc = plsc.load_gather(sqv, [colv])
                d2v = (qn + qc) - 2.0 * (xn * xc + yn * yc + zn * zc)
                return _merge(d2v, colv, rv, ri)

            return lax.cond(jnp.any(cmin < t), proc,
                            lambda a, b_, c_: (a, b_, c_), rv, ri, t)

        rv, ri, _ = lax.fori_loop(0, 16, jstep, (inf16, zi16, inf16))
        idxbuf[pl.ds(r * 16, 16)] = ri + b * W

    pltpu.sync_copy(idxbuf, out_hbm.at[b, pl.ds(row0 * K, ROWS * K)])


def _gather_sc_body(kall, vall, idxg, kf_hbm, vf_hbm, idx_v, krows, vrows,
                    sem1, sem2):
    wid = lax.axis_index("s") * 2 + lax.axis_index("c")
    npw = idxg.shape[0] // 32
    base = wid * npw

    @pl.loop(0, npw // 128)
    def _win(w):
        off = base + w * 128
        pltpu.sync_copy(idxg.at[pl.ds(off, 128)], idx_v)
        c1 = pltpu.async_copy(kall.at[idx_v], krows, sem1)
        c2 = pltpu.async_copy(vall.at[idx_v], vrows, sem2)
        c1.wait()
        c2.wait()
        pltpu.sync_copy(krows, kf_hbm.at[pl.ds(off, 128)])
        pltpu.sync_copy(vrows, vf_hbm.at[pl.ds(off, 128)])


def _gather_sc(kall, vall, idxg):
    N = idxg.shape[0]
    M = kall.shape[0]
    DW = D // 2
    ki = lax.bitcast_convert_type(kall.reshape(M, DW, 2), jnp.int32)
    vi = lax.bitcast_convert_type(vall.reshape(M, DW, 2), jnp.int32)
    mesh = plsc.VectorSubcoreMesh(core_axis_name="c", subcore_axis_name="s")
    cp = pltpu.CompilerParams()
    if "needs_layout_passes" in pltpu.CompilerParams.__dataclass_fields__:
        cp = dataclasses.replace(cp, needs_layout_passes=False)
    k = functools.partial(
        pl.kernel,
        compiler_params=cp,
        out_type=[jax.ShapeDtypeStruct((N, DW), jnp.int32)] * 2,
        mesh=mesh,
        scratch_types=[
            pltpu.VMEM((128,), jnp.int32),
            pltpu.VMEM((128, DW), jnp.int32),
            pltpu.VMEM((128, DW), jnp.int32),
            pltpu.SemaphoreType.DMA,
            pltpu.SemaphoreType.DMA,
        ],
    )(_gather_sc_body)
    kf, vf = k(ki, vi, idxg)
    kf = lax.bitcast_convert_type(kf, jnp.bfloat16).reshape(N, D)
    vf = lax.bitcast_convert_type(vf, jnp.bfloat16).reshape(N, D)
    return kf, vf


def _emb_sc_body(occ_h, lev_h, oct_h, las_h, phi_h, e0_h, e1_h, e2_h, e3_h,
                 e4_h, out_h, occv, levv, octv, lasv, phiv, e0v, e1v, e2v,
                 e3v, e4v, outbuf):
    NT = 2048
    wid = lax.axis_index("s") * 2 + lax.axis_index("c")
    t0 = wid * NT
    pltpu.sync_copy(occ_h.at[pl.ds(t0, NT)], occv)
    pltpu.sync_copy(lev_h.at[pl.ds(t0, NT)], levv)
    pltpu.sync_copy(oct_h.at[pl.ds(t0, NT)], octv)
    pltpu.sync_copy(las_h.at[pl.ds(t0, NT)], lasv)
    pltpu.sync_copy(phi_h.at[pl.ds(t0, NT)], phiv)
    pltpu.sync_copy(e0_h, e0v)
    pltpu.sync_copy(e1_h, e1v)
    pltpu.sync_copy(e2_h, e2v)
    pltpu.sync_copy(e3_h, e3v)
    pltpu.sync_copy(e4_h, e4v)

    @pl.loop(0, NT // 16)
    def _grp(g):
        s = pl.ds(g * 16, 16)
        occ = occv[s] * 24
        lev = levv[s] * 2
        oct_ = octv[s] * 2
        las = lasv[s] * 2
        phi = phiv[s] * 2
        for j in range(24):
            outbuf[pl.ds(j * NT + g * 16, 16)] = plsc.load_gather(
                e0v, [occ + j])
        for j in range(2):
            outbuf[pl.ds((24 + j) * NT + g * 16, 16)] = plsc.load_gather(
                e1v, [lev + j])
            outbuf[pl.ds((26 + j) * NT + g * 16, 16)] = plsc.load_gather(
                e2v, [oct_ + j])
            outbuf[pl.ds((28 + j) * NT + g * 16, 16)] = plsc.load_gather(
                e3v, [las + j])
            outbuf[pl.ds((30 + j) * NT + g * 16, 16)] = plsc.load_gather(
                e4v, [phi + j])

    pltpu.sync_copy(outbuf, out_h.at[wid])


def _emb_sc(occ, lev, oct_, las, phi, e0f, e1f, e2f, e3f, e4f):
    mesh = plsc.VectorSubcoreMesh(core_axis_name="c", subcore_axis_name="s")
    cp = pltpu.CompilerParams()
    if "needs_layout_passes" in pltpu.CompilerParams.__dataclass_fields__:
        cp = dataclasses.replace(cp, needs_layout_passes=False)
    k = functools.partial(
        pl.kernel,
        compiler_params=cp,
        out_type=jax.ShapeDtypeStruct((32, 32 * 2048), jnp.float32),
        mesh=mesh,
        scratch_types=[
            pltpu.VMEM((2048,), jnp.int32),
            pltpu.VMEM((2048,), jnp.int32),
            pltpu.VMEM((2048,), jnp.int32),
            pltpu.VMEM((2048,), jnp.int32),
            pltpu.VMEM((2048,), jnp.int32),
            pltpu.VMEM((e0f.shape[0],), jnp.float32),
            pltpu.VMEM((e1f.shape[0],), jnp.float32),
            pltpu.VMEM((e2f.shape[0],), jnp.float32),
            pltpu.VMEM((e3f.shape[0],), jnp.float32),
            pltpu.VMEM((e4f.shape[0],), jnp.float32),
            pltpu.VMEM((32 * 2048,), jnp.float32),
        ],
    )(_emb_sc_body)
    return k(occ, lev, oct_, las, phi, e0f, e1f, e2f, e3f, e4f)


def _topk_sc(xyzT, sq, cm):
    B = sq.shape[0]
    W = sq.shape[1]
    mesh = plsc.VectorSubcoreMesh(core_axis_name="c", subcore_axis_name="s")
    cp = pltpu.CompilerParams()
    if "needs_layout_passes" in pltpu.CompilerParams.__dataclass_fields__:
        cp = dataclasses.replace(cp, needs_layout_passes=False)
    k = functools.partial(
        pl.kernel,
        compiler_params=cp,
        out_type=jax.ShapeDtypeStruct((B, W * K), jnp.int32),
        mesh=mesh,
        scratch_types=[
            pltpu.VMEM((W,), jnp.float32),
            pltpu.VMEM((W,), jnp.float32),
            pltpu.VMEM((W,), jnp.float32),
            pltpu.VMEM((W,), jnp.float32),
            pltpu.VMEM((512 * (W // 16),), jnp.float32),
            pltpu.VMEM((16,), jnp.int32),
            pltpu.VMEM((16,), jnp.float32),
            pltpu.VMEM((512 * K,), jnp.int32),
        ],
    )(_topk_sc_body)
    return k(xyzT, sq, cm).reshape(B, W, K)


def _qkv_body(feats_ref, wq_ref, wk_ref, wv_ref, q_ref, k_ref, v_ref):
    f = feats_ref[0]
    q_ref[0] = jnp.dot(f, wq_ref[...], preferred_element_type=jnp.float32)
    k_ref[0] = jnp.dot(
        f, wk_ref[...],
        preferred_element_type=jnp.float32).astype(jnp.bfloat16)
    v_ref[0] = jnp.dot(
        f, wv_ref[...],
        preferred_element_type=jnp.float32).astype(jnp.bfloat16)


def _attn_body(q_ref, kf_ref, vf_ref, feats_ref, wo_ref, out_ref):
    q = q_ref[0]
    kf = kf_ref[0].astype(jnp.float32)
    vf = vf_ref[0].astype(jnp.float32)
    logits = jnp.sum(q[:, None, :] * kf, axis=-1) * (1.0 / 16.0)
    m = jnp.max(logits, axis=-1, keepdims=True)
    e = jnp.exp(logits - m)
    attn = e / jnp.sum(e, axis=-1, keepdims=True)
    agg = jnp.sum(attn[:, :, None] * vf, axis=1)
    out_ref[0] = (
        jnp.dot(agg, wo_ref[...], preferred_element_type=jnp.float32)
        + feats_ref[0]
    )


def kernel(occupy, level, octant, laser, phi, pos, enc0, enc1, enc2, enc3,
           enc4, pos_w, Wq, Wk, Wv, Wo):
    W, B = occupy.shape[0], occupy.shape[1]
    e4pad = jnp.concatenate(
        (enc4.reshape(-1), jnp.zeros((4, ), jnp.float32)))
    embf = _emb_sc(occupy.reshape(-1), level.reshape(-1),
                   octant.reshape(-1), laser.reshape(-1), phi.reshape(-1),
                   enc0.reshape(-1), enc1.reshape(-1), enc2.reshape(-1),
                   enc3.reshape(-1), e4pad)
    emb = (embf.reshape(32, 32, 2048).transpose(1, 0, 2)
           .reshape(32, W, B, 4).transpose(1, 2, 3, 0).reshape(W, B, 128))
    pos_min = jnp.min(pos, axis=0, keepdims=True)
    pos_max = jnp.max(pos, axis=0, keepdims=True)
    pos_norm = (pos - pos_min) / (pos_max - pos_min + 1e-07)
    pos128 = pos_norm @ pos_w
    emb = jnp.concatenate((emb, pos128), axis=-1)
    xyz = jnp.transpose(pos_norm, (1, 0, 2))
    feats = jnp.transpose(emb, (1, 0, 2))
    sq = jnp.sum(xyz * xyz, axis=-1)
    xyzT = jnp.transpose(pos_norm, (1, 2, 0)).reshape(B * 3, W)
    cm = _chunkmin_tc(xyzT, sq)
    idxg = _topk_sc(xyzT, sq, cm).reshape(-1)

    qv, kall, vall = pl.pallas_call(
        _qkv_body,
        grid=(B,),
        in_specs=[
            pl.BlockSpec((1, W, D), lambda b: (b, 0, 0)),
            pl.BlockSpec((D, D), lambda b: (0, 0)),
            pl.BlockSpec((D, D), lambda b: (0, 0)),
            pl.BlockSpec((D, D), lambda b: (0, 0)),
        ],
        out_specs=[
            pl.BlockSpec((1, W, D), lambda b: (b, 0, 0)),
            pl.BlockSpec((1, W, D), lambda b: (b, 0, 0)),
            pl.BlockSpec((1, W, D), lambda b: (b, 0, 0)),
        ],
        out_shape=[
            jax.ShapeDtypeStruct((B, W, D), jnp.float32),
            jax.ShapeDtypeStruct((B, W, D), jnp.bfloat16),
            jax.ShapeDtypeStruct((B, W, D), jnp.bfloat16),
        ],
    )(feats, Wq, Wk, Wv)

    kf, vf = _gather_sc(kall.reshape(B * W, D), vall.reshape(B * W, D), idxg)
    kf = kf.reshape(B, W, K, D)
    vf = vf.reshape(B, W, K, D)

    WB = 512
    out = pl.pallas_call(
        _attn_body,
        grid=(B, W // WB),
        in_specs=[
            pl.BlockSpec((1, WB, D), lambda b, w: (b, w, 0)),
            pl.BlockSpec((1, WB, K, D), lambda b, w: (b, w, 0, 0)),
            pl.BlockSpec((1, WB, K, D), lambda b, w: (b, w, 0, 0)),
            pl.BlockSpec((1, WB, D), lambda b, w: (b, w, 0)),
            pl.BlockSpec((D, D), lambda b, w: (0, 0)),
        ],
        out_specs=pl.BlockSpec((1, WB, D), lambda b, w: (b, w, 0)),
        out_shape=jax.ShapeDtypeStruct((B, W, D), jnp.float32),
    )(qv, kf, vf, feats, Wo)
    return out


# speedup vs baseline: 3.3896x; 3.3896x over previous
import dataclasses
import functools

import jax
import jax.numpy as jnp
from jax import lax
from jax.experimental import pallas as pl
from jax.experimental.pallas import tpu as pltpu
from jax.experimental.pallas import tpu_sc as plsc

K = 16
D = 256


def _cm_body(xyzT_ref, sq_ref, xyzr_ref, sqr_ref, cm_ref):
    W = 2048
    RB = 256
    xb = xyzT_ref[0].astype(jnp.bfloat16)
    xrows = xyzr_ref[0].astype(jnp.bfloat16)
    s = lax.dot_general(xrows, xb, (((0,), (0,)), ((), ())),
                        preferred_element_type=jnp.float32)
    sqv = sq_ref[0, 0]
    sqr = sqr_ref[0, 0]
    d2 = (sqr[:, None] + sqv[None, :]) - 2.0 * s
    cm_ref[0] = jnp.min(d2.reshape(RB, W // 16, 16), axis=-1)


def _chunkmin_tc(xyzT, sq):
    B, W = sq.shape
    RB = 256
    return pl.pallas_call(
        _cm_body,
        grid=(B, W // RB),
        in_specs=[
            pl.BlockSpec((1, 3, W), lambda b, i: (b, 0, 0)),
            pl.BlockSpec((1, 1, W), lambda b, i: (b, 0, 0)),
            pl.BlockSpec((1, 3, RB), lambda b, i: (b, 0, i)),
            pl.BlockSpec((1, 1, RB), lambda b, i: (b, 0, i)),
        ],
        out_specs=pl.BlockSpec((1, RB, W // 16), lambda b, i: (b, i, 0)),
        out_shape=jax.ShapeDtypeStruct((B, W, W // 16), jnp.float32),
    )(xyzT.reshape(B, 3, W), sq.reshape(B, 1, W),
      xyzT.reshape(B, 3, W), sq.reshape(B, 1, W)).reshape(B, W * (W // 16))


def _topk_sc_body(xyz_hbm, sq_hbm, cm_hbm, out_hbm, xv, yv, zv, sqv,
                  cmbuf, cmivm, cmvvm, idxbuf):
    W = 2048
    ROWS = 512
    wid = lax.axis_index("s") * 2 + lax.axis_index("c")
    b = wid // 4
    row0 = (wid % 4) * ROWS
    pltpu.sync_copy(xyz_hbm.at[b * 3 + 0], xv)
    pltpu.sync_copy(xyz_hbm.at[b * 3 + 1], yv)
    pltpu.sync_copy(xyz_hbm.at[b * 3 + 2], zv)
    pltpu.sync_copy(sq_hbm.at[b], sqv)
    pltpu.sync_copy(cm_hbm.at[b, pl.ds(row0 * 128, ROWS * 128)], cmbuf)
    iota = lax.iota(jnp.int32, 16)

    @pl.loop(0, W // 16)
    def _rnd(c):
        s = pl.ds(c * 16, 16)
        for ref in (xv, yv, zv):
            u = plsc.bitcast(ref[s], jnp.int32)
            u = (u + 0x7FFF + ((u >> 16) & 1)) & ~0xFFFF
            ref[s] = plsc.bitcast(u, jnp.float32)

    def _merge(d2v, midx, rv, ri):
        cv, ci = plsc.sort_key_val(d2v, midx)
        rcv = lax.rev(cv, (0,))
        rci = lax.rev(ci, (0,))
        m = rv <= rcv
        nv = jnp.where(m, rv, rcv)
        ni = jnp.where(m, ri, rci)
        rv2, ri2 = plsc.sort_key_val(nv, ni)
        t2 = jnp.full((16,), jnp.max(rv2))
        return rv2, ri2, t2

    inf16 = jnp.full((16,), jnp.inf, jnp.float32)
    zi16 = jnp.zeros((16,), jnp.int32)

    @pl.loop(0, ROWS)
    def _row(r):
        s0 = jnp.full((16,), row0 + r, jnp.int32)
        xn = plsc.load_gather(xv, [s0])
        yn = plsc.load_gather(yv, [s0])
        zn = plsc.load_gather(zv, [s0])
        qn = plsc.load_gather(sqv, [s0])

        def cstep(c, carry):
            rv, ri, t = carry
            cmc = cmbuf[pl.ds(r * 128 + c * 16, 16)]

            def m(rv, ri, t):
                return _merge(cmc, iota + c * 16, rv, ri)

            return lax.cond(jnp.any(cmc < t), m,
                            lambda a, b_, c_: (a, b_, c_), rv, ri, t)

        cmv16, cmi16, _ = lax.fori_loop(0, 8, cstep, (inf16, zi16, inf16))
        cmvvm[pl.ds(0, 16)] = cmv16
        cmivm[pl.ds(0, 16)] = cmi16

        def jstep(j, carry):
            rv, ri, t = carry
            js = jnp.full((16,), j, jnp.int32)
            cmin = plsc.load_gather(cmvvm, [js])

            def proc(rv, ri, t):
                cid = plsc.load_gather(cmivm, [js])
                colv = cid * 16 + iota
                xc = plsc.load_gather(xv, [colv])
                yc = plsc.load_gather(yv, [colv])
                zc = plsc.load_gather(zv, [colv])
                qc = plsc.load_gather(sqv, [colv])
                d2v = (qn + qc) - 2.0 * (xn * xc + yn * yc + zn * zc)
                return _merge(d2v, colv, rv, ri)

            return lax.cond(jnp.any(cmin < t), proc,
                            lambda a, b_, c_: (a, b_, c_), rv, ri, t)

        rv, ri, _ = lax.fori_loop(0, 16, jstep, (inf16, zi16, inf16))
        idxbuf[pl.ds(r * 16, 16)] = ri + b * W

    pltpu.sync_copy(idxbuf, out_hbm.at[b, pl.ds(row0 * K, ROWS * K)])


def _gather_sc_body(kall, vall, idxg, kf_hbm, vf_hbm, i0, i1, k0, v0, k1,
                    v1, g0, g1, s0, s1):
    WSZ = 64
    wid = lax.axis_index("s") * 2 + lax.axis_index("c")
    npw = idxg.shape[0] // 32
    base = wid * npw

    @pl.loop(0, npw // (2 * WSZ))
    def _win(p):
        off0 = base + p * (2 * WSZ)
        off1 = off0 + WSZ
        pltpu.sync_copy(idxg.at[pl.ds(off0, WSZ)], i0)
        ck0 = pltpu.async_copy(kall.at[i0], k0, g0)
        cv0 = pltpu.async_copy(vall.at[i0], v0, g0)
        pltpu.sync_copy(idxg.at[pl.ds(off1, WSZ)], i1)
        ck1 = pltpu.async_copy(kall.at[i1], k1, g1)
        cv1 = pltpu.async_copy(vall.at[i1], v1, g1)
        ck0.wait()
        cv0.wait()
        sk0 = pltpu.async_copy(k0, kf_hbm.at[pl.ds(off0, WSZ)], s0)
        sv0 = pltpu.async_copy(v0, vf_hbm.at[pl.ds(off0, WSZ)], s0)
        ck1.wait()
        cv1.wait()
        sk1 = pltpu.async_copy(k1, kf_hbm.at[pl.ds(off1, WSZ)], s1)
        sv1 = pltpu.async_copy(v1, vf_hbm.at[pl.ds(off1, WSZ)], s1)
        sk0.wait()
        sv0.wait()
        sk1.wait()
        sv1.wait()


def _gather_sc(kall, vall, idxg):
    N = idxg.shape[0]
    mesh = plsc.VectorSubcoreMesh(core_axis_name="c", subcore_axis_name="s")
    cp = pltpu.CompilerParams()
    if "needs_layout_passes" in pltpu.CompilerParams.__dataclass_fields__:
        cp = dataclasses.replace(cp, needs_layout_passes=False)
    k = functools.partial(
        pl.kernel,
        compiler_params=cp,
        out_type=[jax.ShapeDtypeStruct((N, D), jnp.float32)] * 2,
        mesh=mesh,
        scratch_types=[
            pltpu.VMEM((64,), jnp.int32),
            pltpu.VMEM((64,), jnp.int32),
            pltpu.VMEM((64, D), jnp.float32),
            pltpu.VMEM((64, D), jnp.float32),
            pltpu.VMEM((64, D), jnp.float32),
            pltpu.VMEM((64, D), jnp.float32),
            pltpu.SemaphoreType.DMA,
            pltpu.SemaphoreType.DMA,
            pltpu.SemaphoreType.DMA,
            pltpu.SemaphoreType.DMA,
        ],
    )(_gather_sc_body)
    return k(kall, vall, idxg)


def _emb_sc_body(occ_h, lev_h, oct_h, las_h, phi_h, e0_h, e1_h, e2_h, e3_h,
                 e4_h, out_h, occv, levv, octv, lasv, phiv, e0v, e1v, e2v,
                 e3v, e4v, outbuf):
    NT = 2048
    wid = lax.axis_index("s") * 2 + lax.axis_index("c")
    t0 = wid * NT
    pltpu.sync_copy(occ_h.at[pl.ds(t0, NT)], occv)
    pltpu.sync_copy(lev_h.at[pl.ds(t0, NT)], levv)
    pltpu.sync_copy(oct_h.at[pl.ds(t0, NT)], octv)
    pltpu.sync_copy(las_h.at[pl.ds(t0, NT)], lasv)
    pltpu.sync_copy(phi_h.at[pl.ds(t0, NT)], phiv)
    pltpu.sync_copy(e0_h, e0v)
    pltpu.sync_copy(e1_h, e1v)
    pltpu.sync_copy(e2_h, e2v)
    pltpu.sync_copy(e3_h, e3v)
    pltpu.sync_copy(e4_h, e4v)

    @pl.loop(0, NT // 16)
    def _grp(g):
        s = pl.ds(g * 16, 16)
        occ = occv[s] * 24
        lev = levv[s] * 2
        oct_ = octv[s] * 2
        las = lasv[s] * 2
        phi = phiv[s] * 2
        for j in range(24):
            outbuf[pl.ds(j * NT + g * 16, 16)] = plsc.load_gather(
                e0v, [occ + j])
        for j in range(2):
            outbuf[pl.ds((24 + j) * NT + g * 16, 16)] = plsc.load_gather(
                e1v, [lev + j])
            outbuf[pl.ds((26 + j) * NT + g * 16, 16)] = plsc.load_gather(
                e2v, [oct_ + j])
            outbuf[pl.ds((28 + j) * NT + g * 16, 16)] = plsc.load_gather(
                e3v, [las + j])
            outbuf[pl.ds((30 + j) * NT + g * 16, 16)] = plsc.load_gather(
                e4v, [phi + j])

    pltpu.sync_copy(outbuf, out_h.at[wid])


def _emb_sc(occ, lev, oct_, las, phi, e0f, e1f, e2f, e3f, e4f):
    mesh = plsc.VectorSubcoreMesh(core_axis_name="c", subcore_axis_name="s")
    cp = pltpu.CompilerParams()
    if "needs_layout_passes" in pltpu.CompilerParams.__dataclass_fields__:
        cp = dataclasses.replace(cp, needs_layout_passes=False)
    k = functools.partial(
        pl.kernel,
        compiler_params=cp,
        out_type=jax.ShapeDtypeStruct((32, 32 * 2048), jnp.float32),
        mesh=mesh,
        scratch_types=[
            pltpu.VMEM((2048,), jnp.int32),
            pltpu.VMEM((2048,), jnp.int32),
            pltpu.VMEM((2048,), jnp.int32),
            pltpu.VMEM((2048,), jnp.int32),
            pltpu.VMEM((2048,), jnp.int32),
            pltpu.VMEM((e0f.shape[0],), jnp.float32),
            pltpu.VMEM((e1f.shape[0],), jnp.float32),
            pltpu.VMEM((e2f.shape[0],), jnp.float32),
            pltpu.VMEM((e3f.shape[0],), jnp.float32),
            pltpu.VMEM((e4f.shape[0],), jnp.float32),
            pltpu.VMEM((32 * 2048,), jnp.float32),
        ],
    )(_emb_sc_body)
    return k(occ, lev, oct_, las, phi, e0f, e1f, e2f, e3f, e4f)


def _topk_sc(xyzT, sq, cm):
    B = sq.shape[0]
    W = sq.shape[1]
    mesh = plsc.VectorSubcoreMesh(core_axis_name="c", subcore_axis_name="s")
    cp = pltpu.CompilerParams()
    if "needs_layout_passes" in pltpu.CompilerParams.__dataclass_fields__:
        cp = dataclasses.replace(cp, needs_layout_passes=False)
    k = functools.partial(
        pl.kernel,
        compiler_params=cp,
        out_type=jax.ShapeDtypeStruct((B, W * K), jnp.int32),
        mesh=mesh,
        scratch_types=[
            pltpu.VMEM((W,), jnp.float32),
            pltpu.VMEM((W,), jnp.float32),
            pltpu.VMEM((W,), jnp.float32),
            pltpu.VMEM((W,), jnp.float32),
            pltpu.VMEM((512 * (W // 16),), jnp.float32),
            pltpu.VMEM((16,), jnp.int32),
            pltpu.VMEM((16,), jnp.float32),
            pltpu.VMEM((512 * K,), jnp.int32),
        ],
    )(_topk_sc_body)
    return k(xyzT, sq, cm).reshape(B, W, K)


def _qkv_body(feats_ref, wq_ref, wk_ref, wv_ref, q_ref, k_ref, v_ref):
    f = feats_ref[0]
    q_ref[0] = jnp.dot(f, wq_ref[...], preferred_element_type=jnp.float32)
    k_ref[0] = jnp.dot(f, wk_ref[...], preferred_element_type=jnp.float32)
    v_ref[0] = jnp.dot(f, wv_ref[...], preferred_element_type=jnp.float32)


def _attn_body(q_ref, kf_ref, vf_ref, feats_ref, wo_ref, out_ref):
    q = q_ref[0]
    kf = kf_ref[0]
    vf = vf_ref[0]
    logits = jnp.sum(q[:, None, :] * kf, axis=-1) * (1.0 / 16.0)
    m = jnp.max(logits, axis=-1, keepdims=True)
    e = jnp.exp(logits - m)
    attn = e / jnp.sum(e, axis=-1, keepdims=True)
    agg = jnp.sum(attn[:, :, None] * vf, axis=1)
    out_ref[0] = (
        jnp.dot(agg, wo_ref[...], preferred_element_type=jnp.float32)
        + feats_ref[0]
    )


def kernel(occupy, level, octant, laser, phi, pos, enc0, enc1, enc2, enc3,
           enc4, pos_w, Wq, Wk, Wv, Wo):
    W, B = occupy.shape[0], occupy.shape[1]
    e4pad = jnp.concatenate(
        (enc4.reshape(-1), jnp.zeros((4, ), jnp.float32)))
    embf = _emb_sc(occupy.reshape(-1), level.reshape(-1),
                   octant.reshape(-1), laser.reshape(-1), phi.reshape(-1),
                   enc0.reshape(-1), enc1.reshape(-1), enc2.reshape(-1),
                   enc3.reshape(-1), e4pad)
    emb = (embf.reshape(32, 32, 2048).transpose(1, 0, 2)
           .reshape(32, W, B, 4).transpose(1, 2, 3, 0).reshape(W, B, 128))
    pos_min = jnp.min(pos, axis=0, keepdims=True)
    pos_max = jnp.max(pos, axis=0, keepdims=True)
    pos_norm = (pos - pos_min) / (pos_max - pos_min + 1e-07)
    pos128 = pos_norm @ pos_w
    emb = jnp.concatenate((emb, pos128), axis=-1)
    xyz = jnp.transpose(pos_norm, (1, 0, 2))
    feats = jnp.transpose(emb, (1, 0, 2))
    sq = jnp.sum(xyz * xyz, axis=-1)
    xyzT = jnp.transpose(pos_norm, (1, 2, 0)).reshape(B * 3, W)
    cm = _chunkmin_tc(xyzT, sq)
    idxg = _topk_sc(xyzT, sq, cm).reshape(-1)

    qv, kall, vall = pl.pallas_call(
        _qkv_body,
        grid=(B,),
        in_specs=[
            pl.BlockSpec((1, W, D), lambda b: (b, 0, 0)),
            pl.BlockSpec((D, D), lambda b: (0, 0)),
            pl.BlockSpec((D, D), lambda b: (0, 0)),
            pl.BlockSpec((D, D), lambda b: (0, 0)),
        ],
        out_specs=[
            pl.BlockSpec((1, W, D), lambda b: (b, 0, 0)),
            pl.BlockSpec((1, W, D), lambda b: (b, 0, 0)),
            pl.BlockSpec((1, W, D), lambda b: (b, 0, 0)),
        ],
        out_shape=[jax.ShapeDtypeStruct((B, W, D), jnp.float32)] * 3,
    )(feats, Wq, Wk, Wv)

    kf, vf = _gather_sc(kall.reshape(B * W, D), vall.reshape(B * W, D), idxg)
    kf = kf.reshape(B, W, K, D)
    vf = vf.reshape(B, W, K, D)

    WB = 512
    out = pl.pallas_call(
        _attn_body,
        grid=(B, W // WB),
        in_specs=[
            pl.BlockSpec((1, WB, D), lambda b, w: (b, w, 0)),
            pl.BlockSpec((1, WB, K, D), lambda b, w: (b, w, 0, 0)),
            pl.BlockSpec((1, WB, K, D), lambda b, w: (b, w, 0, 0)),
            pl.BlockSpec((1, WB, D), lambda b, w: (b, w, 0)),
            pl.BlockSpec((D, D), lambda b, w: (0, 0)),
        ],
        out_specs=pl.BlockSpec((1, WB, D), lambda b, w: (b, w, 0)),
        out_shape=jax.ShapeDtypeStruct((B, W, D), jnp.float32),
    )(qv, kf, vf, feats, Wo)
    return out
